# Initial kernel scaffold; baseline (speedup 1.0000x reference)
import functools

import jax
import jax.numpy as jnp
from jax import lax
from jax.experimental import pallas as pl
from jax.experimental.pallas import tpu as pltpu
from jax.experimental.pallas import tpu_sc as plsc

N = 100000
D = 16
NZ = 64
TROWS = 2 * N + NZ

NW = 32
NODE_SPAN = 3200
NODE_STRIDE = 3120
NCHUNK = 1600
IDXROWS = 13
KCHUNK = 8192


def _table_body(x_ref, fm_ref, fc_ref, w1_ref, b1_ref, w2_ref, b2_ref,
                w3_ref, b3_ref, out_ref):
    i = pl.program_id(0)
    inp = jnp.concatenate([x_ref[...], fm_ref[...]], axis=1)
    h = jnp.maximum(
        jnp.dot(inp, w1_ref[...], preferred_element_type=jnp.float32)
        + b1_ref[...], 0.0)
    h = jnp.maximum(
        jnp.dot(h, w2_ref[...], preferred_element_type=jnp.float32)
        + b2_ref[...], 0.0)
    e = (jnp.dot(h, w3_ref[...], preferred_element_type=jnp.float32)
         + b3_ref[...])

    @pl.when(i < 50)
    def _():
        out_ref[...] = e

    @pl.when((i >= 50) & (i < 100))
    def _():
        out_ref[...] = fc_ref[...]

    @pl.when(i >= 100)
    def _():
        out_ref[...] = jnp.zeros_like(out_ref)


def _build_table(x, feature, w1, b1, w2, b2, w3, b3):
    br = 2000
    grid = (TROWS + br - 1) // br
    return pl.pallas_call(
        _table_body,
        grid=(grid,),
        in_specs=[
            pl.BlockSpec((br, 6), lambda i: (jnp.minimum(i, 49), 0)),
            pl.BlockSpec((br, D), lambda i: (jnp.minimum(i, 49), 0)),
            pl.BlockSpec((br, D), lambda i: (jnp.clip(i - 50, 0, 49), 0)),
            pl.BlockSpec((22, D), lambda i: (0, 0)),
            pl.BlockSpec((1, D), lambda i: (0, 0)),
            pl.BlockSpec((D, D), lambda i: (0, 0)),
            pl.BlockSpec((1, D), lambda i: (0, 0)),
            pl.BlockSpec((D, D), lambda i: (0, 0)),
            pl.BlockSpec((1, D), lambda i: (0, 0)),
        ],
        out_specs=pl.BlockSpec((br, D), lambda i: (i, 0)),
        out_shape=jax.ShapeDtypeStruct((TROWS, D), jnp.float32),
    )(x, feature, feature, w1, b1.reshape(1, D), w2, b2.reshape(1, D),
      w3, b3.reshape(1, D))


def _sc_body(f_hbm, a_hbm, b_hbm, t_hbm, out_hbm,
             winner_v, chunk_v, kidx_v, la_v, lb_v, rows_a_v, rows_b_v,
             sem_a, sem_b):
    cid = lax.axis_index("c")
    sid = lax.axis_index("s")
    w = sid * 2 + cid
    lo = jnp.where(w == NW - 1, N - NODE_SPAN, w * NODE_STRIDE)

    def init_body(i, _):
        winner_v[pl.ds(i * 16, 16)] = jnp.full((16,), -1, jnp.int32)
        return 0
    lax.fori_loop(0, (NODE_SPAN + NZ) // 16, init_body, 0)

    nchunks = 131072 // KCHUNK

    def chunk_body(ci, _):
        pltpu.sync_copy(f_hbm.at[pl.ds(ci * KCHUNK, KCHUNK)], chunk_v)

        def vec_body(vi, _):
            f = chunk_v[pl.ds(vi * 16, 16)]
            m = (f >= lo) & (f < lo + NODE_SPAN)

            @pl.when(jnp.any(m))
            def _():
                kv = ci * KCHUNK + vi * 16 + lax.iota(jnp.int32, 16)
                addr = f - lo
                plsc.store_scatter(winner_v, [addr], kv, mask=m)
                rb = plsc.load_gather(winner_v, [addr], mask=m)
                retry = m & (rb < kv)

                def wcond(rm):
                    return jnp.any(rm)

                def wbody(rm):
                    plsc.store_scatter(winner_v, [addr], kv, mask=rm)
                    rb2 = plsc.load_gather(winner_v, [addr], mask=rm)
                    return rm & (rb2 < kv)

                lax.while_loop(wcond, wbody, retry)
            return 0

        lax.fori_loop(0, KCHUNK // 16, vec_body, 0)
        return 0

    lax.fori_loop(0, nchunks, chunk_body, 0)

    for cc in range(NODE_SPAN // NCHUNK):
        nb = lo + cc * NCHUNK
        base = cc * NCHUNK

        def kidx_body(vi, _):
            wv = winner_v[pl.ds(base + vi * 16, 16)]
            row = vi // 8
            col = (vi % 8) * 16
            kidx_v[row, pl.ds(col, 16)] = jnp.maximum(wv, 0)
            return 0
        lax.fori_loop(0, IDXROWS * 8, kidx_body, 0)

        descs = []
        for j in range(IDXROWS):
            descs.append(pltpu.async_copy(a_hbm.at[kidx_v.at[j]],
                                          la_v.at[j], sem_a))
            descs.append(pltpu.async_copy(b_hbm.at[kidx_v.at[j]],
                                          lb_v.at[j], sem_b))
        for d in descs:
            d.wait()

        def fix_body(vi, _):
            wv = winner_v[pl.ds(base + vi * 16, 16)]
            m = wv >= 0
            node = nb + vi * 16 + lax.iota(jnp.int32, 16)
            row = vi // 8
            col = pl.ds((vi % 8) * 16, 16)
            la = la_v[row, col]
            la_v[row, col] = jnp.where(m, la, N + node)
            lb = lb_v[row, col]
            lb_v[row, col] = jnp.where(m, lb, 2 * N + (node & (NZ - 1)))
            return 0
        lax.fori_loop(0, IDXROWS * 8, fix_body, 0)

        descs = []
        for j in range(IDXROWS):
            descs.append(pltpu.async_copy(t_hbm.at[la_v.at[j]],
                                          rows_a_v.at[pl.ds(j * 128, 128)],
                                          sem_a))
            descs.append(pltpu.async_copy(t_hbm.at[lb_v.at[j]],
                                          rows_b_v.at[pl.ds(j * 128, 128)],
                                          sem_b))
        for d in descs:
            d.wait()

        def add_body(r, _):
            rows_a_v[r, :] = rows_a_v[r, :] + rows_b_v[r, :]
            return 0
        lax.fori_loop(0, NCHUNK, add_body, 0)

        pltpu.sync_copy(rows_a_v.at[pl.ds(0, NCHUNK)],
                        out_hbm.at[pl.ds(nb, NCHUNK)])


@functools.partial(
    pl.kernel,
    out_type=jax.ShapeDtypeStruct((N, D), jnp.float32),
    mesh=plsc.VectorSubcoreMesh(core_axis_name="c", subcore_axis_name="s",
                                num_cores=2, num_subcores=16),
    scratch_types=[
        pltpu.VMEM((NODE_SPAN + NZ,), jnp.int32),
        pltpu.VMEM((KCHUNK,), jnp.int32),
        pltpu.VMEM((IDXROWS, 128), jnp.int32),
        pltpu.VMEM((IDXROWS, 128), jnp.int32),
        pltpu.VMEM((IDXROWS, 128), jnp.int32),
        pltpu.VMEM((IDXROWS * 128, D), jnp.float32),
        pltpu.VMEM((IDXROWS * 128, D), jnp.float32),
        pltpu.SemaphoreType.DMA,
        pltpu.SemaphoreType.DMA,
    ],
)
def _sc_resolve(f_hbm, a_hbm, b_hbm, t_hbm, out_hbm, *scratch):
    _sc_body(f_hbm, a_hbm, b_hbm, t_hbm, out_hbm, *scratch)


def kernel(X, Feature, I_list, W1, b1, W2, b2, W3, b3):
    tri = I_list[:, 0, :, :]
    a_list = tri[..., 0].reshape(-1).astype(jnp.int32)
    b_list = tri[..., 1].reshape(-1).astype(jnp.int32)
    fathers = tri[..., 2].reshape(-1).astype(jnp.int32)
    table = _build_table(X, Feature, W1, b1, W2, b2, W3, b3)
    return _sc_resolve(fathers, a_list, b_list, table)


# speedup vs baseline: 3.9372x; 3.9372x over previous
import functools

import jax
import jax.numpy as jnp
from jax import lax
from jax.experimental import pallas as pl
from jax.experimental.pallas import tpu as pltpu
from jax.experimental.pallas import tpu_sc as plsc

N = 100000
D = 16
NZ = 64
TROWS = 2 * N + NZ

NW = 32
NODE_SPAN = 3200
NODE_STRIDE = 3120
NCHUNK = 1600
IDXROWS = 13
KCHUNK = 8192


def _table_body(x_ref, fm_ref, fc_ref, w1_ref, b1_ref, w2_ref, b2_ref,
                w3_ref, b3_ref, out_ref):
    i = pl.program_id(0)
    inp = jnp.concatenate([x_ref[...], fm_ref[...]], axis=1)
    h = jnp.maximum(
        jnp.dot(inp, w1_ref[...], preferred_element_type=jnp.float32)
        + b1_ref[...], 0.0)
    h = jnp.maximum(
        jnp.dot(h, w2_ref[...], preferred_element_type=jnp.float32)
        + b2_ref[...], 0.0)
    e = (jnp.dot(h, w3_ref[...], preferred_element_type=jnp.float32)
         + b3_ref[...])

    @pl.when(i < 50)
    def _():
        out_ref[...] = e

    @pl.when((i >= 50) & (i < 100))
    def _():
        out_ref[...] = fc_ref[...]

    @pl.when(i >= 100)
    def _():
        out_ref[...] = jnp.zeros_like(out_ref)


def _build_table(x, feature, w1, b1, w2, b2, w3, b3):
    br = 2000
    grid = (TROWS + br - 1) // br
    return pl.pallas_call(
        _table_body,
        grid=(grid,),
        in_specs=[
            pl.BlockSpec((br, 6), lambda i: (jnp.minimum(i, 49), 0)),
            pl.BlockSpec((br, D), lambda i: (jnp.minimum(i, 49), 0)),
            pl.BlockSpec((br, D), lambda i: (jnp.clip(i - 50, 0, 49), 0)),
            pl.BlockSpec((22, D), lambda i: (0, 0)),
            pl.BlockSpec((1, D), lambda i: (0, 0)),
            pl.BlockSpec((D, D), lambda i: (0, 0)),
            pl.BlockSpec((1, D), lambda i: (0, 0)),
            pl.BlockSpec((D, D), lambda i: (0, 0)),
            pl.BlockSpec((1, D), lambda i: (0, 0)),
        ],
        out_specs=pl.BlockSpec((br, D), lambda i: (i, 0)),
        out_shape=jax.ShapeDtypeStruct((TROWS, D), jnp.float32),
    )(x, feature, feature, w1, b1.reshape(1, D), w2, b2.reshape(1, D),
      w3, b3.reshape(1, D))


def _sc_body(f_hbm, a_hbm, b_hbm, t_hbm, out_hbm,
             winner_v, chunk_v, kidx_v, la_v, lb_v, rows_a_v, rows_b_v,
             sem_a, sem_b):
    cid = lax.axis_index("c")
    sid = lax.axis_index("s")
    w = sid * 2 + cid
    lo = jnp.where(w == NW - 1, N - NODE_SPAN, w * NODE_STRIDE)

    def init_body(i, _):
        winner_v[pl.ds(i * 16, 16)] = jnp.full((16,), -1, jnp.int32)
        return 0
    lax.fori_loop(0, (NODE_SPAN + NZ) // 16, init_body, 0)

    nchunks = 131072 // KCHUNK

    def chunk_body(ci, _):
        pltpu.sync_copy(f_hbm.at[pl.ds(ci * KCHUNK, KCHUNK)], chunk_v)

        def vec_body(vi, _):
            f = chunk_v[pl.ds(vi * 16, 16)]
            m = (f >= lo) & (f < lo + NODE_SPAN)

            @pl.when(plsc.all_reduce_population_count(m)[0] > 0)
            def _():
                kv = ci * KCHUNK + vi * 16 + lax.iota(jnp.int32, 16)
                addr = f - lo
                plsc.store_scatter(winner_v, [addr], kv, mask=m)
                rb = plsc.load_gather(winner_v, [addr], mask=m)
                retry = m & (rb < kv)

                def wcond(rm):
                    return plsc.all_reduce_population_count(rm)[0] > 0

                def wbody(rm):
                    plsc.store_scatter(winner_v, [addr], kv, mask=rm)
                    rb2 = plsc.load_gather(winner_v, [addr], mask=rm)
                    return rm & (rb2 < kv)

                lax.while_loop(wcond, wbody, retry)
            return 0

        lax.fori_loop(0, KCHUNK // 16, vec_body, 0)
        return 0

    lax.fori_loop(0, nchunks, chunk_body, 0)

    for cc in range(NODE_SPAN // NCHUNK):
        nb = lo + cc * NCHUNK
        base = cc * NCHUNK

        def kidx_body(vi, _):
            wv = winner_v[pl.ds(base + vi * 16, 16)]
            row = vi // 8
            col = (vi % 8) * 16
            kidx_v[row, pl.ds(col, 16)] = jnp.maximum(wv, 0)
            return 0
        lax.fori_loop(0, IDXROWS * 8, kidx_body, 0)

        descs = []
        for j in range(IDXROWS):
            descs.append(pltpu.async_copy(a_hbm.at[kidx_v.at[j]],
                                          la_v.at[j], sem_a))
            descs.append(pltpu.async_copy(b_hbm.at[kidx_v.at[j]],
                                          lb_v.at[j], sem_b))
        for d in descs:
            d.wait()

        def fix_body(vi, _):
            wv = winner_v[pl.ds(base + vi * 16, 16)]
            m = wv >= 0
            node = nb + vi * 16 + lax.iota(jnp.int32, 16)
            row = vi // 8
            col = pl.ds((vi % 8) * 16, 16)
            la = la_v[row, col]
            la_v[row, col] = jnp.where(m, la, N + node)
            lb = lb_v[row, col]
            lb_v[row, col] = jnp.where(m, lb, 2 * N + (node & (NZ - 1)))
            return 0
        lax.fori_loop(0, IDXROWS * 8, fix_body, 0)

        descs = []
        for j in range(IDXROWS):
            descs.append(pltpu.async_copy(t_hbm.at[la_v.at[j]],
                                          rows_a_v.at[pl.ds(j * 128, 128)],
                                          sem_a))
            descs.append(pltpu.async_copy(t_hbm.at[lb_v.at[j]],
                                          rows_b_v.at[pl.ds(j * 128, 128)],
                                          sem_b))
        for d in descs:
            d.wait()

        def add_body(r, _):
            rows_a_v[r, :] = rows_a_v[r, :] + rows_b_v[r, :]
            return 0
        lax.fori_loop(0, NCHUNK, add_body, 0)

        pltpu.sync_copy(rows_a_v.at[pl.ds(0, NCHUNK)],
                        out_hbm.at[pl.ds(nb, NCHUNK)])


@functools.partial(
    pl.kernel,
    out_type=jax.ShapeDtypeStruct((N, D), jnp.float32),
    mesh=plsc.VectorSubcoreMesh(core_axis_name="c", subcore_axis_name="s",
                                num_cores=2, num_subcores=16),
    compiler_params=pltpu.CompilerParams(needs_layout_passes=False,
                                         use_tc_tiling_on_sc=False),
    scratch_types=[
        pltpu.VMEM((NODE_SPAN + NZ,), jnp.int32),
        pltpu.VMEM((KCHUNK,), jnp.int32),
        pltpu.VMEM((IDXROWS, 128), jnp.int32),
        pltpu.VMEM((IDXROWS, 128), jnp.int32),
        pltpu.VMEM((IDXROWS, 128), jnp.int32),
        pltpu.VMEM((IDXROWS * 128, D), jnp.float32),
        pltpu.VMEM((IDXROWS * 128, D), jnp.float32),
        pltpu.SemaphoreType.DMA,
        pltpu.SemaphoreType.DMA,
    ],
)
def _sc_resolve(f_hbm, a_hbm, b_hbm, t_hbm, out_hbm, *scratch):
    _sc_body(f_hbm, a_hbm, b_hbm, t_hbm, out_hbm, *scratch)


def kernel(X, Feature, I_list, W1, b1, W2, b2, W3, b3):
    tri = I_list[:, 0, :, :]
    a_list = tri[..., 0].reshape(-1).astype(jnp.int32)
    b_list = tri[..., 1].reshape(-1).astype(jnp.int32)
    fathers = tri[..., 2].reshape(-1).astype(jnp.int32)
    table = _build_table(X, Feature, W1, b1, W2, b2, W3, b3)
    return _sc_resolve(fathers, a_list, b_list, table)


# speedup vs baseline: 6.0545x; 1.5378x over previous
import functools

import jax
import jax.numpy as jnp
from jax import lax
from jax.experimental import pallas as pl
from jax.experimental.pallas import tpu as pltpu
from jax.experimental.pallas import tpu_sc as plsc

N = 100000
NPAD = 100096
D = 16
NZ = 64
TROWS = 2 * N + NZ
NK = 131072

NW = 32
KSLICE = NK // NW
NODE_SPAN = 3200
NODE_STRIDE = 3120
NCHUNK = 1600
IDXROWS = 13
CSPAN = IDXROWS * 128

_SC_PARAMS = pltpu.CompilerParams(needs_layout_passes=False,
                                  use_tc_tiling_on_sc=False)


def _table_body(x_ref, fm_ref, fc_ref, w1_ref, b1_ref, w2_ref, b2_ref,
                w3_ref, b3_ref, out_ref):
    i = pl.program_id(0)
    inp = jnp.concatenate([x_ref[...], fm_ref[...]], axis=1)
    h = jnp.maximum(
        jnp.dot(inp, w1_ref[...], preferred_element_type=jnp.float32)
        + b1_ref[...], 0.0)
    h = jnp.maximum(
        jnp.dot(h, w2_ref[...], preferred_element_type=jnp.float32)
        + b2_ref[...], 0.0)
    e = (jnp.dot(h, w3_ref[...], preferred_element_type=jnp.float32)
         + b3_ref[...])

    @pl.when(i < 50)
    def _():
        out_ref[...] = e

    @pl.when((i >= 50) & (i < 100))
    def _():
        out_ref[...] = fc_ref[...]

    @pl.when(i >= 100)
    def _():
        out_ref[...] = jnp.zeros_like(out_ref)


def _build_table(x, feature, w1, b1, w2, b2, w3, b3):
    br = 2000
    grid = (TROWS + br - 1) // br
    return pl.pallas_call(
        _table_body,
        grid=(grid,),
        in_specs=[
            pl.BlockSpec((br, 6), lambda i: (jnp.minimum(i, 49), 0)),
            pl.BlockSpec((br, D), lambda i: (jnp.minimum(i, 49), 0)),
            pl.BlockSpec((br, D), lambda i: (jnp.clip(i - 50, 0, 49), 0)),
            pl.BlockSpec((22, D), lambda i: (0, 0)),
            pl.BlockSpec((1, D), lambda i: (0, 0)),
            pl.BlockSpec((D, D), lambda i: (0, 0)),
            pl.BlockSpec((1, D), lambda i: (0, 0)),
            pl.BlockSpec((D, D), lambda i: (0, 0)),
            pl.BlockSpec((1, D), lambda i: (0, 0)),
        ],
        out_specs=pl.BlockSpec((br, D), lambda i: (i, 0)),
        out_shape=jax.ShapeDtypeStruct((TROWS, D), jnp.float32),
    )(x, feature, feature, w1, b1.reshape(1, D), w2, b2.reshape(1, D),
      w3, b3.reshape(1, D))


def _winner_body(f_hbm, part_hbm, wloc_v, chunk_v):
    cid = lax.axis_index("c")
    sid = lax.axis_index("s")
    w = sid * 2 + cid
    kbase = w * KSLICE

    def init_body(i, _):
        neg = jnp.full((16,), -1, jnp.int32)
        for u in range(8):
            wloc_v[pl.ds(i * 128 + u * 16, 16)] = neg
        return 0
    lax.fori_loop(0, NPAD // 128, init_body, 0)

    pltpu.sync_copy(f_hbm.at[pl.ds(kbase, KSLICE)], chunk_v)

    lane = lax.iota(jnp.int32, 16)
    nxt_idx = jnp.minimum(lane + 1, 15).reshape(16, 1)
    gdn = lax.GatherDimensionNumbers(offset_dims=(),
                                     collapsed_slice_dims=(0,),
                                     start_index_map=(0,))

    def vec_body(vi, _):
        f = chunk_v[pl.ds(vi * 16, 16)]
        kv = kbase + vi * 16 + lane
        key = f * 16 + lane
        ks, vs = plsc.sort_key_val(key, kv)
        fs = lax.shift_right_arithmetic(ks, 4)
        nxt = lax.gather(fs, nxt_idx, gdn, (1,),
                         mode=lax.GatherScatterMode.PROMISE_IN_BOUNDS)
        keep = (fs != nxt) | (lane == 15)
        plsc.store_scatter(wloc_v, [fs], vs, mask=keep)
        return 0

    lax.fori_loop(0, KSLICE // 16, vec_body, 0)
    pltpu.sync_copy(wloc_v, part_hbm.at[w])


def _resolve_body(a_hbm, b_hbm, t_hbm, part_hbm, out_hbm,
                  winner_v, mbuf_v, kidx_v, la_v, lb_v, rows_a_v, rows_b_v,
                  sem_a, sem_b):
    cid = lax.axis_index("c")
    sid = lax.axis_index("s")
    w = sid * 2 + cid
    lo = jnp.where(w == NW - 1, N - NODE_SPAN, w * NODE_STRIDE)

    for cc in range(NODE_SPAN // NCHUNK):
        nb = lo + cc * NCHUNK

        pltpu.sync_copy(part_hbm.at[0, pl.ds(nb, CSPAN)], winner_v)
        for j in range(1, NW):
            pltpu.sync_copy(part_hbm.at[j, pl.ds(nb, CSPAN)], mbuf_v)

            def merge_body(vi, _):
                sl = pl.ds(vi * 16, 16)
                winner_v[sl] = jnp.maximum(winner_v[sl], mbuf_v[sl])
                return 0
            lax.fori_loop(0, CSPAN // 16, merge_body, 0)

        def kidx_body(vi, _):
            wv = winner_v[pl.ds(vi * 16, 16)]
            row = vi // 8
            col = (vi % 8) * 16
            kidx_v[row, pl.ds(col, 16)] = jnp.maximum(wv, 0)
            return 0
        lax.fori_loop(0, IDXROWS * 8, kidx_body, 0)

        descs = []
        for j in range(IDXROWS):
            descs.append(pltpu.async_copy(a_hbm.at[kidx_v.at[j]],
                                          la_v.at[j], sem_a))
            descs.append(pltpu.async_copy(b_hbm.at[kidx_v.at[j]],
                                          lb_v.at[j], sem_b))
        for d in descs:
            d.wait()

        def fix_body(vi, _):
            wv = winner_v[pl.ds(vi * 16, 16)]
            m = wv >= 0
            node = nb + vi * 16 + lax.iota(jnp.int32, 16)
            row = vi // 8
            col = pl.ds((vi % 8) * 16, 16)
            la = la_v[row, col]
            la_v[row, col] = jnp.where(m, la, N + node)
            lb = lb_v[row, col]
            lb_v[row, col] = jnp.where(m, lb, 2 * N + (node & (NZ - 1)))
            return 0
        lax.fori_loop(0, IDXROWS * 8, fix_body, 0)

        descs = []
        for j in range(IDXROWS):
            descs.append(pltpu.async_copy(t_hbm.at[la_v.at[j]],
                                          rows_a_v.at[pl.ds(j * 128, 128)],
                                          sem_a))
            descs.append(pltpu.async_copy(t_hbm.at[lb_v.at[j]],
                                          rows_b_v.at[pl.ds(j * 128, 128)],
                                          sem_b))
        for d in descs:
            d.wait()

        def add_body(r, _):
            for u in range(4):
                rr = r * 4 + u
                rows_a_v[rr, :] = rows_a_v[rr, :] + rows_b_v[rr, :]
            return 0
        lax.fori_loop(0, NCHUNK // 4, add_body, 0)

        pltpu.sync_copy(rows_a_v.at[pl.ds(0, NCHUNK)],
                        out_hbm.at[pl.ds(nb, NCHUNK)])


def _make_sc_kernels():
    mesh = plsc.VectorSubcoreMesh(core_axis_name="c", subcore_axis_name="s",
                                  num_cores=2, num_subcores=16)
    winner_partials = pl.kernel(
        _winner_body,
        out_type=jax.ShapeDtypeStruct((NW, NPAD), jnp.int32),
        mesh=mesh,
        compiler_params=_SC_PARAMS,
        scratch_types=[
            pltpu.VMEM((NPAD,), jnp.int32),
            pltpu.VMEM((KSLICE,), jnp.int32),
        ],
    )
    resolve = pl.kernel(
        _resolve_body,
        out_type=jax.ShapeDtypeStruct((N, D), jnp.float32),
        mesh=mesh,
        compiler_params=_SC_PARAMS,
        scratch_types=[
            pltpu.VMEM((CSPAN,), jnp.int32),
            pltpu.VMEM((CSPAN,), jnp.int32),
            pltpu.VMEM((IDXROWS, 128), jnp.int32),
            pltpu.VMEM((IDXROWS, 128), jnp.int32),
            pltpu.VMEM((IDXROWS, 128), jnp.int32),
            pltpu.VMEM((CSPAN, D), jnp.float32),
            pltpu.VMEM((CSPAN, D), jnp.float32),
            pltpu.SemaphoreType.DMA,
            pltpu.SemaphoreType.DMA,
        ],
    )
    return winner_partials, resolve


def kernel(X, Feature, I_list, W1, b1, W2, b2, W3, b3):
    tri = I_list[:, 0, :, :]
    a_list = tri[..., 0].reshape(-1).astype(jnp.int32)
    b_list = tri[..., 1].reshape(-1).astype(jnp.int32)
    fathers = tri[..., 2].reshape(-1).astype(jnp.int32)
    winner_partials, resolve = _make_sc_kernels()
    table = _build_table(X, Feature, W1, b1, W2, b2, W3, b3)
    partials = winner_partials(fathers)
    return resolve(a_list, b_list, table, partials)


# speedup vs baseline: 8.2768x; 1.3671x over previous
import functools

import jax
import jax.numpy as jnp
from jax import lax
from jax.experimental import pallas as pl
from jax.experimental.pallas import tpu as pltpu
from jax.experimental.pallas import tpu_sc as plsc

N = 100000
NPAD = 100096
D = 16
NZ = 64
TROWS = 2 * N + NZ
NK = 131072

NW = 32
NP = 8
KSLICE = NK // NP
NODE_SPAN = 3200
NODE_STRIDE = 3120
NCHUNK = 1600
IDXROWS = 13
CSPAN = IDXROWS * 128

_SC_PARAMS = pltpu.CompilerParams(needs_layout_passes=False,
                                  use_tc_tiling_on_sc=False)


def _table_body(x8_ref, f8_ref, w1x_ref, w1f_ref, b1_ref,
                w2_ref, b2_ref, w3_ref, b3_ref, out_ref):
    h = jnp.maximum(
        jnp.dot(x8_ref[...], w1x_ref[...],
                preferred_element_type=jnp.float32)
        + jnp.dot(f8_ref[...], w1f_ref[...],
                  preferred_element_type=jnp.float32)
        + b1_ref[...], 0.0)
    h = jnp.maximum(
        jnp.dot(h, w2_ref[...], preferred_element_type=jnp.float32)
        + b2_ref[...], 0.0)
    out_ref[...] = (jnp.dot(h, w3_ref[...],
                            preferred_element_type=jnp.float32)
                    + b3_ref[...])


def _build_table(x, feature, w1, b1, w2, b2, w3, b3):
    x8 = x.reshape(12500, 48)
    f8 = feature.reshape(12500, 128)
    eye8 = jnp.eye(8, dtype=jnp.float32)
    w1x = jnp.kron(eye8, w1[:6])
    w1f = jnp.kron(eye8, w1[6:])
    w2_8 = jnp.kron(eye8, w2)
    w3_8 = jnp.kron(eye8, w3)
    b1_8 = jnp.tile(b1, 8).reshape(1, 128)
    b2_8 = jnp.tile(b2, 8).reshape(1, 128)
    b3_8 = jnp.tile(b3, 8).reshape(1, 128)
    e8 = pl.pallas_call(
        _table_body,
        out_shape=jax.ShapeDtypeStruct((12500, 128), jnp.float32),
    )(x8, f8, w1x, w1f, b1_8, w2_8, b2_8, w3_8, b3_8)
    flat = jnp.concatenate([e8.reshape(-1), feature.reshape(-1),
                            jnp.zeros(NZ * D, jnp.float32)])
    return flat.reshape(TROWS, D)


def _winner_body(f_hbm, part_hbm, wloc_v, chunk_v):
    cid = lax.axis_index("c")
    sid = lax.axis_index("s")
    w = sid * 2 + cid

    @pl.when(w < NP)
    def _():
        kbase = w * KSLICE

        def init_body(i, _):
            neg = jnp.full((16,), -1, jnp.int32)
            for u in range(8):
                wloc_v[pl.ds(i * 128 + u * 16, 16)] = neg
            return 0
        lax.fori_loop(0, NPAD // 128, init_body, 0)

        pltpu.sync_copy(f_hbm.at[pl.ds(kbase, KSLICE)], chunk_v)

        lane = lax.iota(jnp.int32, 16)
        nxt_idx = jnp.minimum(lane + 1, 15).reshape(16, 1)
        gdn = lax.GatherDimensionNumbers(offset_dims=(),
                                         collapsed_slice_dims=(0,),
                                         start_index_map=(0,))

        def vec_body(vi, _):
            f = chunk_v[pl.ds(vi * 16, 16)]
            kv = kbase + vi * 16 + lane
            key = f * 16 + lane
            ks, vs = plsc.sort_key_val(key, kv)
            fs = lax.shift_right_arithmetic(ks, 4)
            nxt = lax.gather(fs, nxt_idx, gdn, (1,),
                             mode=lax.GatherScatterMode.PROMISE_IN_BOUNDS)
            keep = (fs != nxt) | (lane == 15)
            plsc.store_scatter(wloc_v, [fs], vs, mask=keep)
            return 0

        lax.fori_loop(0, KSLICE // 16, vec_body, 0)
        pltpu.sync_copy(wloc_v, part_hbm.at[w])


def _resolve_body(a_hbm, b_hbm, t_hbm, part_hbm, out_hbm,
                  winner_v, mbuf_v, kidx_v, la_v, lb_v, rows_a_v, rows_b_v,
                  sem_a, sem_b):
    cid = lax.axis_index("c")
    sid = lax.axis_index("s")
    w = sid * 2 + cid
    lo = jnp.where(w == NW - 1, N - NODE_SPAN, w * NODE_STRIDE)

    for cc in range(NODE_SPAN // NCHUNK):
        nb = lo + cc * NCHUNK

        pltpu.sync_copy(part_hbm.at[0, pl.ds(nb, CSPAN)], winner_v)
        for j in range(1, NP):
            pltpu.sync_copy(part_hbm.at[j, pl.ds(nb, CSPAN)], mbuf_v)

            def merge_body(vi, _):
                sl = pl.ds(vi * 16, 16)
                winner_v[sl] = jnp.maximum(winner_v[sl], mbuf_v[sl])
                return 0
            lax.fori_loop(0, CSPAN // 16, merge_body, 0)

        def kidx_body(vi, _):
            wv = winner_v[pl.ds(vi * 16, 16)]
            row = vi // 8
            col = (vi % 8) * 16
            kidx_v[row, pl.ds(col, 16)] = jnp.maximum(wv, 0)
            return 0
        lax.fori_loop(0, IDXROWS * 8, kidx_body, 0)

        descs = []
        for j in range(IDXROWS):
            descs.append(pltpu.async_copy(a_hbm.at[kidx_v.at[j]],
                                          la_v.at[j], sem_a))
            descs.append(pltpu.async_copy(b_hbm.at[kidx_v.at[j]],
                                          lb_v.at[j], sem_b))
        for d in descs:
            d.wait()

        def fix_body(vi, _):
            wv = winner_v[pl.ds(vi * 16, 16)]
            m = wv >= 0
            node = nb + vi * 16 + lax.iota(jnp.int32, 16)
            row = vi // 8
            col = pl.ds((vi % 8) * 16, 16)
            la = la_v[row, col]
            la_v[row, col] = jnp.where(m, la, N + node)
            lb = lb_v[row, col]
            lb_v[row, col] = jnp.where(m, lb, 2 * N + (node & (NZ - 1)))
            return 0
        lax.fori_loop(0, IDXROWS * 8, fix_body, 0)

        descs = []
        for j in range(IDXROWS):
            descs.append(pltpu.async_copy(t_hbm.at[la_v.at[j]],
                                          rows_a_v.at[pl.ds(j * 128, 128)],
                                          sem_a))
            descs.append(pltpu.async_copy(t_hbm.at[lb_v.at[j]],
                                          rows_b_v.at[pl.ds(j * 128, 128)],
                                          sem_b))
        for d in descs:
            d.wait()

        def add_body(r, _):
            for u in range(4):
                rr = r * 4 + u
                rows_a_v[rr, :] = rows_a_v[rr, :] + rows_b_v[rr, :]
            return 0
        lax.fori_loop(0, NCHUNK // 4, add_body, 0)

        pltpu.sync_copy(rows_a_v.at[pl.ds(0, NCHUNK)],
                        out_hbm.at[pl.ds(nb, NCHUNK)])


def _make_sc_kernels():
    mesh = plsc.VectorSubcoreMesh(core_axis_name="c", subcore_axis_name="s",
                                  num_cores=2, num_subcores=16)
    winner_partials = pl.kernel(
        _winner_body,
        out_type=jax.ShapeDtypeStruct((NP, NPAD), jnp.int32),
        mesh=mesh,
        compiler_params=_SC_PARAMS,
        scratch_types=[
            pltpu.VMEM((NPAD,), jnp.int32),
            pltpu.VMEM((KSLICE,), jnp.int32),
        ],
    )
    resolve = pl.kernel(
        _resolve_body,
        out_type=jax.ShapeDtypeStruct((N, D), jnp.float32),
        mesh=mesh,
        compiler_params=_SC_PARAMS,
        scratch_types=[
            pltpu.VMEM((CSPAN,), jnp.int32),
            pltpu.VMEM((CSPAN,), jnp.int32),
            pltpu.VMEM((IDXROWS, 128), jnp.int32),
            pltpu.VMEM((IDXROWS, 128), jnp.int32),
            pltpu.VMEM((IDXROWS, 128), jnp.int32),
            pltpu.VMEM((CSPAN, D), jnp.float32),
            pltpu.VMEM((CSPAN, D), jnp.float32),
            pltpu.SemaphoreType.DMA,
            pltpu.SemaphoreType.DMA,
        ],
    )
    return winner_partials, resolve


def kernel(X, Feature, I_list, W1, b1, W2, b2, W3, b3):
    tri = I_list[:, 0, :, :]
    a_list = tri[..., 0].reshape(-1).astype(jnp.int32)
    b_list = tri[..., 1].reshape(-1).astype(jnp.int32)
    fathers = tri[..., 2].reshape(-1).astype(jnp.int32)
    winner_partials, resolve = _make_sc_kernels()
    table = _build_table(X, Feature, W1, b1, W2, b2, W3, b3)
    partials = winner_partials(fathers)
    return resolve(a_list, b_list, table, partials)
